# baseline (speedup 1.0000x reference)
import functools

import jax
import jax.numpy as jnp
from jax import lax
from jax.experimental import pallas as pl
from jax.experimental.pallas import tpu as pltpu
from jax.experimental.pallas import tpu_sc as plsc

_NUM_CORES = 2
_NUM_SUBCORES = 16
_NW = _NUM_CORES * _NUM_SUBCORES
_LANES = 16


def _make_pool_kernel(B, Lseq, D):
    rows_per_w = B // _NW
    n_streams = Lseq
    nsub = D // _LANES
    mesh = plsc.VectorSubcoreMesh(
        core_axis_name="c", subcore_axis_name="s",
        num_cores=_NUM_CORES, num_subcores=_NUM_SUBCORES)

    @functools.partial(
        pl.kernel,
        out_type=jax.ShapeDtypeStruct((B, D), jnp.float32),
        mesh=mesh,
        scratch_types=[
            pltpu.VMEM((n_streams, rows_per_w), jnp.int32),
            pltpu.VMEM((2, rows_per_w), jnp.int32),
            pltpu.VMEM((4, rows_per_w, D), jnp.float32),
            pltpu.VMEM((rows_per_w, D), jnp.float32),
            pltpu.VMEM_SHARED(
                (2 * _NUM_SUBCORES * rows_per_w, D), jnp.float32),
            [pltpu.SemaphoreType.DMA] * 4,
            [pltpu.SemaphoreType.DMA] * 4,
        ],
    )
    def pool(tok_hbm, sidx_hbm, emb_hbm, out_hbm,
             idx_v, sidx_v, rows_v, comb_v, pooled_sh, gsem, ssem):
        sid = lax.axis_index("s")
        wid = sid * _NUM_CORES + lax.axis_index("c")
        base_row = wid * rows_per_w
        pltpu.sync_copy(tok_hbm.at[wid], idx_v)
        pltpu.sync_copy(sidx_hbm.at[sid], sidx_v)

        def g_start(k, b):
            pltpu.async_copy(emb_hbm.at[idx_v.at[k]], rows_v.at[b], gsem[b])

        def g_wait(k, b):
            pltpu.make_async_copy(
                emb_hbm.at[idx_v.at[k]], rows_v.at[b], gsem[b]).wait()

        def s_start(k, b, par, add):
            pltpu.async_copy(
                rows_v.at[b], pooled_sh.at[sidx_v.at[par]], ssem[b], add=add)

        def s_wait(k, b, par):
            pltpu.make_async_copy(
                rows_v.at[b], pooled_sh.at[sidx_v.at[par]], ssem[b]).wait()

        g_start(0, 0)
        g_start(1, 1)

        def visit(k, b, par, warm, cool, add=True):
            g_wait(k, b)
            s_start(k, b, par, add)
            b2 = (b + 2) % 4
            if warm:
                s_wait(k - 2, b2, par)
            if cool:
                g_start(k + 2, b2)

        visit(0, 0, 0, warm=False, cool=True, add=False)
        visit(1, 1, 1, warm=False, cool=True, add=False)

        nquad = (n_streams - 6) // 4

        def quad(i, carry):
            k = 4 * i + 2
            for q in range(4):
                visit(k + q, (2 + q) % 4, q % 2, warm=True, cool=True)
            return carry
        lax.fori_loop(0, nquad, quad, 0)

        for kk in range(2 + 4 * nquad, n_streams):
            visit(kk, kk % 4, kk % 2, warm=True, cool=kk + 2 < n_streams)
        for kk in range(n_streams - 2, n_streams):
            s_wait(kk, kk % 4, kk % 2)

        pltpu.sync_copy(
            pooled_sh.at[pl.ds(sid * rows_per_w, rows_per_w)], rows_v.at[0])
        pltpu.sync_copy(
            pooled_sh.at[pl.ds((_NUM_SUBCORES + sid) * rows_per_w,
                               rows_per_w)], rows_v.at[1])

        def cbody(r, carry):
            for c in range(nsub):
                sl = pl.ds(c * _LANES, _LANES)
                comb_v[r, sl] = rows_v[0, r, sl] + rows_v[1, r, sl]
            return carry
        lax.fori_loop(0, rows_per_w, cbody, 0)
        pltpu.sync_copy(comb_v, out_hbm.at[pl.ds(base_row, rows_per_w)])

    return pool


def _project(pooled, W, b, scale):
    B, D = pooled.shape
    M = W.shape[0]
    BLK = 512

    def mm(x_ref, w_ref, b_ref, o_ref):
        o_ref[...] = lax.dot_general(
            x_ref[...], w_ref[...], (((1,), (1,)), ((), ())),
            preferred_element_type=jnp.float32) * scale + b_ref[...]

    return pl.pallas_call(
        mm,
        grid=(B // BLK,),
        in_specs=[
            pl.BlockSpec((BLK, D), lambda i: (i, 0)),
            pl.BlockSpec((M, D), lambda i: (0, 0)),
            pl.BlockSpec((1, M), lambda i: (0, 0)),
        ],
        out_specs=pl.BlockSpec((BLK, M), lambda i: (i, 0)),
        out_shape=jax.ShapeDtypeStruct((B, M), jnp.float32),
    )(pooled, W, b.reshape(1, M))


def kernel(token_ids, emb, W, b):
    B, Lseq = token_ids.shape
    rows_per_w = B // _NW
    tok = token_ids.astype(jnp.int32).reshape(
        _NW, rows_per_w, Lseq).transpose(0, 2, 1)
    lanes = jnp.arange(rows_per_w, dtype=jnp.int32)
    sids = jnp.arange(_NUM_SUBCORES, dtype=jnp.int32)
    pars = jnp.arange(2, dtype=jnp.int32)
    sidx = (pars[None, :, None] * (_NUM_SUBCORES * rows_per_w)
            + sids[:, None, None] * rows_per_w + lanes[None, None, :])
    pooled = _make_pool_kernel(B, Lseq, emb.shape[1])(tok, sidx, emb)
    out = _project(pooled, W, b, 1.0 / Lseq)
    return out[:, None, :]


# speedup vs baseline: 1.3094x; 1.3094x over previous
import functools

import jax
import jax.numpy as jnp
from jax import lax
from jax.experimental import pallas as pl
from jax.experimental.pallas import tpu as pltpu
from jax.experimental.pallas import tpu_sc as plsc

_NUM_CORES = 2
_NUM_SUBCORES = 16
_NW = _NUM_CORES * _NUM_SUBCORES
_LANES = 16
_UNROLL = 5


def _make_pool_kernel(B, Lseq, D):
    rows_per_w = B // _NW
    CR = 2
    chunk_len = CR * Lseq
    n_chunks = rows_per_w // CR
    nsub = D // _LANES
    scale = 1.0 / Lseq
    mesh = plsc.VectorSubcoreMesh(
        core_axis_name="c", subcore_axis_name="s",
        num_cores=_NUM_CORES, num_subcores=_NUM_SUBCORES)

    @functools.partial(
        pl.kernel,
        out_type=jax.ShapeDtypeStruct((B, D), jnp.float32),
        mesh=mesh,
        scratch_types=[
            pltpu.VMEM((n_chunks, chunk_len), jnp.int32),
            pltpu.VMEM((4, chunk_len, D), jnp.float32),
            pltpu.VMEM((rows_per_w, D), jnp.float32),
            [pltpu.SemaphoreType.DMA] * 4,
        ],
    )
    def pool(tok_hbm, emb_hbm, out_hbm, idx_v, rows_v, pooled_v, gsem):
        wid = lax.axis_index("s") * _NUM_CORES + lax.axis_index("c")
        base_row = wid * rows_per_w
        pltpu.sync_copy(tok_hbm.at[wid], idx_v)

        def start(chunk, b):
            pltpu.async_copy(emb_hbm.at[idx_v.at[chunk]], rows_v.at[b],
                             gsem[b])

        def wait(chunk, b):
            pltpu.make_async_copy(
                emb_hbm.at[idx_v.at[chunk]], rows_v.at[b], gsem[b]).wait()

        def accumulate(chunk, b):
            for r in range(CR):
                def body(t, accs, r=r):
                    base = r * Lseq + t * _UNROLL
                    for u in range(_UNROLL):
                        accs = tuple(
                            accs[c] + rows_v[b, base + u,
                                             pl.ds(c * _LANES, _LANES)]
                            for c in range(nsub))
                    return accs
                accs = lax.fori_loop(
                    0, Lseq // _UNROLL, body,
                    tuple(jnp.zeros((_LANES,), jnp.float32)
                          for _ in range(nsub)))
                row = chunk * CR + r
                for c in range(nsub):
                    pooled_v[row, pl.ds(c * _LANES, _LANES)] = accs[c] * scale

        start(0, 0)
        start(1, 1)
        start(2, 2)

        def visit(k, b, cool):
            wait(k, b)
            if cool:
                start(k + 3, (b + 3) % 4)
            accumulate(k, b)

        nquad = (n_chunks - 3) // 4

        def quad(i, carry):
            k = 4 * i
            for q in range(4):
                visit(k + q, q, cool=True)
            return carry
        lax.fori_loop(0, nquad, quad, 0)

        for kk in range(4 * nquad, n_chunks):
            visit(kk, kk % 4, cool=kk + 3 < n_chunks)

        pltpu.sync_copy(pooled_v, out_hbm.at[pl.ds(base_row, rows_per_w)])

    return pool


def _project(pooled, W, b):
    B, D = pooled.shape
    M = W.shape[0]
    BLK = 512

    def mm(x_ref, w_ref, b_ref, o_ref):
        o_ref[...] = lax.dot_general(
            x_ref[...], w_ref[...], (((1,), (1,)), ((), ())),
            preferred_element_type=jnp.float32) + b_ref[...]

    return pl.pallas_call(
        mm,
        grid=(B // BLK,),
        in_specs=[
            pl.BlockSpec((BLK, D), lambda i: (i, 0)),
            pl.BlockSpec((M, D), lambda i: (0, 0)),
            pl.BlockSpec((1, M), lambda i: (0, 0)),
        ],
        out_specs=pl.BlockSpec((BLK, M), lambda i: (i, 0)),
        out_shape=jax.ShapeDtypeStruct((B, M), jnp.float32),
    )(pooled, W, b.reshape(1, M))


def kernel(token_ids, emb, W, b):
    B, Lseq = token_ids.shape
    idx_per_w = (B // _NW) * Lseq
    chunk = 2 * Lseq
    tok = token_ids.astype(jnp.int32).reshape(
        _NW, idx_per_w // chunk, chunk)
    pooled = _make_pool_kernel(B, Lseq, emb.shape[1])(tok, emb)
    out = _project(pooled, W, b)
    return out[:, None, :]
